# baseline (speedup 1.0000x reference)
import jax
import jax.numpy as jnp
from jax import lax
from jax.experimental import pallas as pl
from jax.experimental.pallas import tpu as pltpu
from jax.experimental.pallas import tpu_sc as plsc

D = 64
L = 16
NC = 2
NS = 16
NW = NC * NS


def _body(u_hbm, i_hbm, U_hbm, V_hbm, out_hbm,
          idx_u, idx_i, rows_u, rows_v, out_v, sem_u, sem_v):
    B = out_hbm.shape[0]
    bpw = B // NW
    wid = lax.axis_index("s") * NC + lax.axis_index("c")
    base = wid * bpw

    pltpu.sync_copy(u_hbm.at[pl.ds(base, bpw)], idx_u)
    pltpu.sync_copy(i_hbm.at[pl.ds(base, bpw)], idx_i)
    cu = pltpu.async_copy(U_hbm.at[idx_u], rows_u, sem_u)
    cv = pltpu.async_copy(V_hbm.at[idx_i], rows_v, sem_v)
    cu.wait()
    cv.wait()

    lane = lax.iota(jnp.int32, L)

    def group(g, carry):
        b0 = g * L
        acc = jnp.zeros((L,), jnp.float32)
        for k in range(L):
            r = b0 + k
            s = rows_u[r, pl.ds(0, L)] * rows_v[r, pl.ds(0, L)]
            for c in range(1, D // L):
                s = s + rows_u[r, pl.ds(c * L, L)] * rows_v[r, pl.ds(c * L, L)]
            acc = jnp.where(lane == k, jnp.sum(s), acc)
        out_v[pl.ds(b0, L)] = acc
        return carry

    lax.fori_loop(0, bpw // L, group, 0)
    pltpu.sync_copy(out_v, out_hbm.at[pl.ds(base, bpw)])


def kernel(u, i, U_emb, V_emb):
    B = u.shape[0]
    bpw = B // NW
    mesh = plsc.VectorSubcoreMesh(core_axis_name="c", subcore_axis_name="s")
    f = pl.kernel(
        _body,
        out_type=jax.ShapeDtypeStruct((B,), jnp.float32),
        mesh=mesh,
        compiler_params=pltpu.CompilerParams(
            needs_layout_passes=False, use_tc_tiling_on_sc=False),
        scratch_types=[
            pltpu.VMEM((bpw,), jnp.int32),
            pltpu.VMEM((bpw,), jnp.int32),
            pltpu.VMEM((bpw, D), jnp.float32),
            pltpu.VMEM((bpw, D), jnp.float32),
            pltpu.VMEM((bpw,), jnp.float32),
            pltpu.SemaphoreType.DMA,
            pltpu.SemaphoreType.DMA,
        ],
    )
    return f(u.astype(jnp.int32), i.astype(jnp.int32), U_emb, V_emb)


# speedup vs baseline: 2.1553x; 2.1553x over previous
import jax
import jax.numpy as jnp
from jax import lax
from jax.experimental import pallas as pl
from jax.experimental.pallas import tpu as pltpu
from jax.experimental.pallas import tpu_sc as plsc

D = 64
L = 16
NC = 2
NS = 16
NW = NC * NS
C = 16


def _body(u_hbm, i_hbm, U_hbm, V_hbm, out_hbm,
          idx_u, idx_i, sub_u, sub_i, slabs_u, slabs_v, out_v,
          sem_u, sem_v):
    B = out_hbm.shape[0]
    bpw = B // NW
    wid = lax.axis_index("s") * NC + lax.axis_index("c")
    base = wid * bpw

    pltpu.sync_copy(u_hbm.at[pl.ds(base, bpw)], idx_u)
    pltpu.sync_copy(i_hbm.at[pl.ds(base, bpw)], idx_i)

    def split(k, carry):
        o = k * L
        raw_u = idx_u[pl.ds(o, L)]
        raw_i = idx_i[pl.ds(o, L)]
        idx_u[pl.ds(o, L)] = raw_u >> 3
        idx_i[pl.ds(o, L)] = raw_i >> 3
        sub_u[pl.ds(o, L)] = raw_u & 7
        sub_i[pl.ds(o, L)] = raw_i & 7
        return carry

    lax.fori_loop(0, bpw // L, split, 0)

    lane = lax.iota(jnp.int32, L)

    def chunk(g, carry):
        b0 = g * C
        slabv_u = idx_u[pl.ds(b0, C)]
        slabv_i = idx_i[pl.ds(b0, C)]
        descs = []
        for k in range(C):
            descs.append(
                pltpu.async_copy(U_hbm.at[slabv_u[k]], slabs_u.at[k], sem_u))
            descs.append(
                pltpu.async_copy(V_hbm.at[slabv_i[k]], slabs_v.at[k], sem_v))
        for dd in descs:
            dd.wait()
        acc = jnp.zeros((L,), jnp.float32)
        subv_u = sub_u[pl.ds(b0, C)]
        subv_i = sub_i[pl.ds(b0, C)]
        for k in range(C):
            su = subv_u[k]
            si = subv_i[k]
            s = slabs_u[k, su, pl.ds(0, L)] * slabs_v[k, si, pl.ds(0, L)]
            for c in range(1, D // L):
                s = s + (slabs_u[k, su, pl.ds(c * L, L)]
                         * slabs_v[k, si, pl.ds(c * L, L)])
            acc = jnp.where(lane == k, jnp.sum(s), acc)
        out_v[pl.ds(b0, C)] = acc
        return carry

    lax.fori_loop(0, bpw // C, chunk, 0)
    pltpu.sync_copy(out_v, out_hbm.at[pl.ds(base, bpw)])


def kernel(u, i, U_emb, V_emb):
    B = u.shape[0]
    bpw = B // NW
    n_user = U_emb.shape[0]
    n_item = V_emb.shape[0]
    Ur = U_emb.reshape(n_user // 8, 8, D)
    Vr = V_emb.reshape(n_item // 8, 8, D)
    mesh = plsc.VectorSubcoreMesh(core_axis_name="c", subcore_axis_name="s")
    f = pl.kernel(
        _body,
        out_type=jax.ShapeDtypeStruct((B,), jnp.float32),
        mesh=mesh,
        compiler_params=pltpu.CompilerParams(
            needs_layout_passes=False, use_tc_tiling_on_sc=True),
        scratch_types=[
            pltpu.VMEM((bpw,), jnp.int32),
            pltpu.VMEM((bpw,), jnp.int32),
            pltpu.VMEM((bpw,), jnp.int32),
            pltpu.VMEM((bpw,), jnp.int32),
            pltpu.VMEM((C, 8, D), jnp.float32),
            pltpu.VMEM((C, 8, D), jnp.float32),
            pltpu.VMEM((bpw,), jnp.float32),
            pltpu.SemaphoreType.DMA,
            pltpu.SemaphoreType.DMA,
        ],
    )
    return f(u.astype(jnp.int32), i.astype(jnp.int32), Ur, Vr)
